# baseline (speedup 1.0000x reference)
import functools

import jax
import jax.numpy as jnp
from jax import lax
from jax.experimental import pallas as pl
from jax.experimental.pallas import tpu as pltpu
from jax.experimental.pallas import tpu_sc as plsc


def kernel(nodes, table):
    (B,) = nodes.shape
    V, D = table.shape
    L = 16

    info = plsc.get_sparse_core_info()
    NC, NS = info.num_cores, info.num_subcores
    NW = NC * NS
    b_per_w = B // NW
    CH = 32
    n_ch = b_per_w // CH

    TR = D // 8
    TC_ALL = B // 128
    tc_per_w = TC_ALL // NW

    mesh = plsc.VectorSubcoreMesh(core_axis_name="c", subcore_axis_name="s")

    @functools.partial(
        pl.kernel,
        mesh=mesh,
        out_type=jax.ShapeDtypeStruct((TR, TC_ALL, 8, 128), jnp.float32),
        scratch_types=[
            pltpu.VMEM((b_per_w,), jnp.int32),
            pltpu.VMEM((2, CH, 8, D), jnp.float32),
            pltpu.VMEM((TR, tc_per_w, 8, 128), jnp.float32),
            pltpu.SemaphoreType.DMA,
            pltpu.SemaphoreType.DMA,
        ],
        compiler_params=pltpu.CompilerParams(
            use_tc_tiling_on_sc=True, needs_layout_passes=False
        ),
    )
    def emb(table_hbm, idx_hbm, out_hbm, idx_v, rows_v, stage_v, sem0, sem1):
        wid = lax.axis_index("s") * NC + lax.axis_index("c")
        base = wid * b_per_w
        pltpu.sync_copy(idx_hbm.at[pl.ds(base, b_per_w)], idx_v)

        lane = lax.iota(jnp.int32, L)
        ge8 = lax.shift_right_logical(lane, 3)
        dsub = lane & 7
        d1 = [2 * k + ge8 for k in range(D // L)]

        def fire(c, buf, sem):
            for m in range(CH // L):
                iv = idx_v[pl.ds(c * CH + m * L, L)]
                gv = lax.shift_right_logical(iv, 3) * 8
                for l in range(L):
                    g = pl.multiple_of(gv[l], 8)
                    pltpu.async_copy(
                        table_hbm.at[pl.ds(g, 8)],
                        rows_v.at[buf, m * L + l],
                        sem,
                    )

        def drain(buf, sem):
            pltpu.make_async_copy(
                table_hbm.at[pl.ds(0, CH * 8)],
                rows_v.at[buf].reshape(CH * 8, D),
                sem,
            ).wait()

        def extract(c, buf):
            for m in range(CH // L):
                iv = idx_v[pl.ds(c * CH + m * L, L)]
                sv = iv & 7
                for l in range(L):
                    s = sv[l]
                    j = c * CH + m * L + l
                    jc_v = jnp.full((L,), j // 128, jnp.int32)
                    jj_v = jnp.full((L,), j % 128, jnp.int32)
                    for k in range(D // L):
                        row16 = rows_v[buf, m * L + l, s, pl.ds(k * L, L)]
                        plsc.store_scatter(
                            stage_v, [d1[k], jc_v, dsub, jj_v], row16
                        )

        fire(0, 0, sem0)

        def pair(c2, _):
            c0 = c2 * 2
            fire(c0 + 1, 1, sem1)
            drain(0, sem0)
            extract(c0, 0)

            @pl.when(c0 + 2 < n_ch)
            def _fire_next():
                fire(c0 + 2, 0, sem0)

            drain(1, sem1)
            extract(c0 + 1, 1)
            return _

        lax.fori_loop(0, n_ch // 2, pair, 0)
        pltpu.sync_copy(stage_v, out_hbm.at[:, pl.ds(wid * tc_per_w, tc_per_w)])

    idx = nodes.astype(jnp.int32)
    t_rm = table.at[0, 0].set(table[0, 0])
    out4 = emb(t_rm, idx)
    return out4.transpose(1, 3, 0, 2).reshape(B, D)
